# baseline (speedup 1.0000x reference)
import functools

import jax
import jax.numpy as jnp
from jax import lax
from jax.experimental import pallas as pl
from jax.experimental.pallas import tpu as pltpu
from jax.experimental.pallas import tpu_sc as plsc

N_TOTAL = 16777216
N_BINS = 15
NC, NS, L = 2, 16, 16
NW = NC * NS

ROWS, COLS = 16384, 1024
BLK_R = 256
N_BLKS = ROWS // BLK_R
K_TC = 19
K_SC = N_BLKS - K_TC

N_SC = K_SC * BLK_R * COLS
PER_W = N_SC // NW
CH = 8192
NCH = PER_W // CH
NBUF = 3


def _sc_body(conf_hbm, pred_hbm, lab_hbm, out_hbm,
             conf0, pred0, lab0, conf1, pred1, lab1, conf2, pred2, lab2,
             tabc, tabf, taba, red, sem0, sem1, sem2):
    wid = lax.axis_index("s") * NC + lax.axis_index("c")
    base = wid * PER_W

    zero = jnp.zeros((L,), jnp.float32)
    for t in (tabc, tabf, taba):
        for r in range(L):
            t[pl.ds(r * L, L)] = zero

    lane = lax.iota(jnp.int32, L)
    ones = jnp.full((L,), 1.0, jnp.float32)

    bufs = ((conf0, pred0, lab0, sem0), (conf1, pred1, lab1, sem1),
            (conf2, pred2, lab2, sem2))

    def start(c, b):
        off = base + c * CH
        cb, pb, lb, sem = bufs[b]
        pltpu.async_copy(conf_hbm.at[pl.ds(off, CH)], cb, sem)
        pltpu.async_copy(pred_hbm.at[pl.ds(off, CH)], pb, sem)
        pltpu.async_copy(lab_hbm.at[pl.ds(off, CH)], lb, sem)

    def wait(b):
        cb, pb, lb, sem = bufs[b]
        pltpu.make_async_copy(conf_hbm.at[pl.ds(0, CH)], cb, sem).wait()
        pltpu.make_async_copy(pred_hbm.at[pl.ds(0, CH)], pb, sem).wait()
        pltpu.make_async_copy(lab_hbm.at[pl.ds(0, CH)], lb, sem).wait()

    def process(b):
        cb, pb, lb, _ = bufs[b]

        @plsc.parallel_loop(0, CH, L, unroll=8)
        def body(s):
            c = cb[pl.ds(s, L)]
            p = pb[pl.ds(s, L)]
            lbl = lb[pl.ds(s, L)]
            addr = (c * 15.0).astype(jnp.int32) * L + lane
            plsc.addupdate_scatter(tabc, [addr], ones)
            plsc.addupdate_scatter(tabf, [addr], c)
            plsc.addupdate_scatter(taba, [addr], ones, mask=p == lbl)

    for c in range(NBUF - 1):
        start(c, c)

    def outer(k, carry):
        c0 = k * NBUF
        for j in range(NBUF):
            c = c0 + j
            b = j
            wait(b)

            @pl.when(c + NBUF - 1 < NCH)
            def _():
                start(c + NBUF - 1, (b + NBUF - 1) % NBUF)

            process(b)
        return carry

    lax.fori_loop(0, NCH // NBUF, outer, 0)
    for c in range((NCH // NBUF) * NBUF, NCH):
        b = c % NBUF
        wait(b)
        process(b)

    for q, t in enumerate((tabc, tabf, taba)):
        acc_v = jnp.zeros((L,), jnp.float32)
        for k in range(L):
            acc_v = acc_v + plsc.load_gather(t, [lane * L + k])
        red[q] = acc_v
    pltpu.sync_copy(red, out_hbm.at[wid])


_sc_hist = functools.partial(
    pl.kernel,
    mesh=plsc.VectorSubcoreMesh(
        core_axis_name="c", subcore_axis_name="s",
        num_cores=NC, num_subcores=NS),
    out_type=jax.ShapeDtypeStruct((NW, 3, L), jnp.float32),
    compiler_params=pltpu.CompilerParams(needs_layout_passes=False),
    scratch_types=(
        [pltpu.VMEM((CH,), jnp.float32),
         pltpu.VMEM((CH,), jnp.int32),
         pltpu.VMEM((CH,), jnp.int32)] * NBUF
        + [pltpu.VMEM((L * L,), jnp.float32)] * 3
        + [pltpu.VMEM((3, L), jnp.float32)]
        + [pltpu.SemaphoreType.DMA] * NBUF
    ),
)(_sc_body)


def _tc_body(c_ref, p_ref, l_ref, o_ref):
    g = pl.program_id(0)

    @pl.when(g == 0)
    def _():
        o_ref[...] = jnp.zeros((3, L), jnp.float32)

    c = c_ref[...]
    acc = (p_ref[...] == l_ref[...]).astype(jnp.float32)
    bi = (c * 15.0).astype(jnp.int32)
    cnts, sconfs, saccs = [], [], []
    for i in range(N_BINS):
        m = bi == i
        mf = jnp.where(m, 1.0, 0.0)
        cnts.append(jnp.sum(mf))
        sconfs.append(jnp.sum(jnp.where(m, c, 0.0)))
        saccs.append(jnp.sum(jnp.where(m, acc, 0.0)))
    part = jnp.stack(
        [jnp.stack(q + [jnp.float32(0.0)]) for q in (cnts, sconfs, saccs)])
    o_ref[...] += part


_tc_hist = pl.pallas_call(
    _tc_body,
    grid=(K_TC,),
    in_specs=[pl.BlockSpec((BLK_R, COLS), lambda g: (K_SC + g, 0))] * 3,
    out_specs=pl.BlockSpec((3, L), lambda g: (0, 0)),
    out_shape=jax.ShapeDtypeStruct((3, L), jnp.float32),
)


def _fin_body(p_ref, o_ref):
    cnt = jnp.sum(p_ref[0], axis=0, keepdims=True)
    sconf = jnp.sum(p_ref[1], axis=0, keepdims=True)
    sacc = jnp.sum(p_ref[2], axis=0, keepdims=True)
    denom = jnp.maximum(cnt, 1.0)
    contrib = jnp.abs(sconf / denom - sacc / denom) * (cnt / N_TOTAL)
    contrib = jnp.where(cnt > 0, contrib, 0.0)
    o_ref[0] = jnp.sum(contrib)


_finalize = pl.pallas_call(
    _fin_body,
    out_shape=jax.ShapeDtypeStruct((1,), jnp.float32),
    out_specs=pl.BlockSpec(memory_space=pltpu.SMEM),
)


def kernel(confidences, predictions, labels, title):
    conf2d = confidences.reshape(ROWS, COLS)
    pred2d = predictions.reshape(ROWS, COLS)
    lab2d = labels.reshape(ROWS, COLS)
    sc_part = _sc_hist(confidences, predictions, labels)
    tc_part = _tc_hist(conf2d, pred2d, lab2d)
    allp = jnp.concatenate(
        [jnp.transpose(sc_part, (1, 0, 2)), tc_part[:, None, :]], axis=1)
    return _finalize(allp)


# speedup vs baseline: 1.2576x; 1.2576x over previous
import functools

import jax
import jax.numpy as jnp
from jax import lax
from jax.experimental import pallas as pl
from jax.experimental.pallas import tpu as pltpu
from jax.experimental.pallas import tpu_sc as plsc

N_TOTAL = 16777216
N_BINS = 15
NC, NS, L = 2, 16, 16
NW = NC * NS

ROWS, COLS = 16384, 1024
BLK_R = 256
N_BLKS = ROWS // BLK_R
K_TC = 4
K_SC = N_BLKS - K_TC

N_SC = K_SC * BLK_R * COLS
PER_W = N_SC // NW
CH = 8192
NCH = PER_W // CH
NBUF = 3


def _sc_body(conf_hbm, pred_hbm, lab_hbm, out_hbm,
             conf0, pred0, lab0, conf1, pred1, lab1, conf2, pred2, lab2,
             tabc, tabf, taba, red, sem0, sem1, sem2):
    wid = lax.axis_index("s") * NC + lax.axis_index("c")
    base = wid * PER_W

    zero = jnp.zeros((L,), jnp.float32)
    for t in (tabc, tabf, taba):
        for r in range(L):
            t[pl.ds(r * L, L)] = zero

    lane = lax.iota(jnp.int32, L)
    ones = jnp.full((L,), 1.0, jnp.float32)

    bufs = ((conf0, pred0, lab0, sem0), (conf1, pred1, lab1, sem1),
            (conf2, pred2, lab2, sem2))

    def start(c, b):
        off = base + c * CH
        cb, pb, lb, sem = bufs[b]
        pltpu.async_copy(conf_hbm.at[pl.ds(off, CH)], cb, sem)
        pltpu.async_copy(pred_hbm.at[pl.ds(off, CH)], pb, sem)
        pltpu.async_copy(lab_hbm.at[pl.ds(off, CH)], lb, sem)

    def wait(b):
        cb, pb, lb, sem = bufs[b]
        pltpu.make_async_copy(conf_hbm.at[pl.ds(0, CH)], cb, sem).wait()
        pltpu.make_async_copy(pred_hbm.at[pl.ds(0, CH)], pb, sem).wait()
        pltpu.make_async_copy(lab_hbm.at[pl.ds(0, CH)], lb, sem).wait()

    def process(b):
        cb, pb, lb, _ = bufs[b]

        @plsc.parallel_loop(0, CH, L, unroll=8)
        def body(s):
            c = cb[pl.ds(s, L)]
            p = pb[pl.ds(s, L)]
            lbl = lb[pl.ds(s, L)]
            addr = (c * 15.0).astype(jnp.int32) * L + lane
            plsc.addupdate_scatter(tabc, [addr], ones)
            plsc.addupdate_scatter(tabf, [addr], c)
            plsc.addupdate_scatter(taba, [addr], ones, mask=p == lbl)

    for c in range(NBUF - 1):
        start(c, c)

    def outer(k, carry):
        c0 = k * NBUF
        for j in range(NBUF):
            c = c0 + j
            b = j
            wait(b)

            @pl.when(c + NBUF - 1 < NCH)
            def _():
                start(c + NBUF - 1, (b + NBUF - 1) % NBUF)

            process(b)
        return carry

    lax.fori_loop(0, NCH // NBUF, outer, 0)
    for c in range((NCH // NBUF) * NBUF, NCH):
        b = c % NBUF
        wait(b)
        process(b)

    for q, t in enumerate((tabc, tabf, taba)):
        acc_v = jnp.zeros((L,), jnp.float32)
        for k in range(L):
            acc_v = acc_v + plsc.load_gather(t, [lane * L + k])
        red[q] = acc_v
    pltpu.sync_copy(red, out_hbm.at[wid])


_sc_hist = functools.partial(
    pl.kernel,
    mesh=plsc.VectorSubcoreMesh(
        core_axis_name="c", subcore_axis_name="s",
        num_cores=NC, num_subcores=NS),
    out_type=jax.ShapeDtypeStruct((NW, 3, L), jnp.float32),
    compiler_params=pltpu.CompilerParams(needs_layout_passes=False),
    scratch_types=(
        [pltpu.VMEM((CH,), jnp.float32),
         pltpu.VMEM((CH,), jnp.int32),
         pltpu.VMEM((CH,), jnp.int32)] * NBUF
        + [pltpu.VMEM((L * L,), jnp.float32)] * 3
        + [pltpu.VMEM((3, L), jnp.float32)]
        + [pltpu.SemaphoreType.DMA] * NBUF
    ),
)(_sc_body)


def _tc_body(c_ref, p_ref, l_ref, o_ref):
    g = pl.program_id(0)

    @pl.when(g == 0)
    def _():
        o_ref[...] = jnp.zeros((3 * L, COLS), jnp.float32)

    c = c_ref[...]
    acc = (p_ref[...] == l_ref[...]).astype(jnp.float32)
    bi = (c * 15.0).astype(jnp.int32)
    vals = (None, c, acc)
    rows = []
    for q in range(3):
        for i in range(N_BINS):
            m = bi == i
            v = jnp.where(m, 1.0, 0.0) if q == 0 else jnp.where(m, vals[q], 0.0)
            rows.append(jnp.sum(v, axis=0, keepdims=True))
        rows.append(jnp.zeros((1, COLS), jnp.float32))
    o_ref[...] += jnp.concatenate(rows, axis=0)


_tc_hist = pl.pallas_call(
    _tc_body,
    grid=(K_TC,),
    in_specs=[pl.BlockSpec((BLK_R, COLS), lambda g: (K_SC + g, 0))] * 3,
    out_specs=pl.BlockSpec((3 * L, COLS), lambda g: (0, 0)),
    out_shape=jax.ShapeDtypeStruct((3 * L, COLS), jnp.float32),
)


def _fin_body(p_ref, t_ref, o_ref):
    def tot(q):
        sc = jnp.sum(p_ref[q], axis=0, keepdims=True)
        tc = jnp.sum(t_ref[q * L:(q + 1) * L, :], axis=1, keepdims=True)
        return sc + jnp.transpose(tc)

    cnt = tot(0)
    sconf = tot(1)
    sacc = tot(2)
    denom = jnp.maximum(cnt, 1.0)
    contrib = jnp.abs(sconf / denom - sacc / denom) * (cnt / N_TOTAL)
    contrib = jnp.where(cnt > 0, contrib, 0.0)
    o_ref[0] = jnp.sum(contrib)


_finalize = pl.pallas_call(
    _fin_body,
    out_shape=jax.ShapeDtypeStruct((1,), jnp.float32),
    out_specs=pl.BlockSpec(memory_space=pltpu.SMEM),
)


def kernel(confidences, predictions, labels, title):
    conf2d = confidences.reshape(ROWS, COLS)
    pred2d = predictions.reshape(ROWS, COLS)
    lab2d = labels.reshape(ROWS, COLS)
    sc_part = _sc_hist(confidences, predictions, labels)
    tc_part = _tc_hist(conf2d, pred2d, lab2d)
    return _finalize(jnp.transpose(sc_part, (1, 0, 2)), tc_part)


# speedup vs baseline: 2.7873x; 2.2164x over previous
import functools

import jax
import jax.numpy as jnp
from jax import lax
from jax.experimental import pallas as pl
from jax.experimental.pallas import tpu as pltpu
from jax.experimental.pallas import tpu_sc as plsc

N_TOTAL = 16777216
N_BINS = 15
NC, NS, L = 2, 16, 16
NW = NC * NS
PER_W = N_TOTAL // NW
CH = 8192
NCH = PER_W // CH
NBUF = 3


def _sc_body(conf_hbm, pred_hbm, lab_hbm, out_hbm,
             conf0, pred0, lab0, conf1, pred1, lab1, conf2, pred2, lab2,
             tabca, tabf, red, sem0, sem1, sem2):
    wid = lax.axis_index("s") * NC + lax.axis_index("c")
    base = wid * PER_W

    zero = jnp.zeros((L,), jnp.float32)
    zero_i = jnp.zeros((L,), jnp.int32)
    for r in range(L):
        tabca[pl.ds(r * L, L)] = zero_i
        tabf[pl.ds(r * L, L)] = zero

    lane = lax.iota(jnp.int32, L)
    one_i = jnp.full((L,), 1, jnp.int32)
    one_acc = jnp.full((L,), 65537, jnp.int32)

    bufs = ((conf0, pred0, lab0, sem0), (conf1, pred1, lab1, sem1),
            (conf2, pred2, lab2, sem2))

    def start(c, b):
        off = base + c * CH
        cb, pb, lb, sem = bufs[b]
        pltpu.async_copy(conf_hbm.at[pl.ds(off, CH)], cb, sem)
        pltpu.async_copy(pred_hbm.at[pl.ds(off, CH)], pb, sem)
        pltpu.async_copy(lab_hbm.at[pl.ds(off, CH)], lb, sem)

    def wait(b):
        cb, pb, lb, sem = bufs[b]
        pltpu.make_async_copy(conf_hbm.at[pl.ds(0, CH)], cb, sem).wait()
        pltpu.make_async_copy(pred_hbm.at[pl.ds(0, CH)], pb, sem).wait()
        pltpu.make_async_copy(lab_hbm.at[pl.ds(0, CH)], lb, sem).wait()

    def process(b):
        cb, pb, lb, _ = bufs[b]

        @plsc.parallel_loop(0, CH, L, unroll=16)
        def body(s):
            c = cb[pl.ds(s, L)]
            p = pb[pl.ds(s, L)]
            lbl = lb[pl.ds(s, L)]
            addr = (c * 15.0).astype(jnp.int32) * L + lane
            ca = jnp.where(p == lbl, one_acc, one_i)
            plsc.addupdate_scatter(tabca, [addr], ca)
            plsc.addupdate_scatter(tabf, [addr], c)

    for c in range(NBUF - 1):
        start(c, c)

    def outer(k, carry):
        c0 = k * NBUF
        for j in range(NBUF):
            c = c0 + j
            b = j
            wait(b)

            @pl.when(c + NBUF - 1 < NCH)
            def _():
                start(c + NBUF - 1, (b + NBUF - 1) % NBUF)

            process(b)
        return carry

    lax.fori_loop(0, NCH // NBUF, outer, 0)
    for c in range((NCH // NBUF) * NBUF, NCH):
        b = c % NBUF
        wait(b)
        process(b)

    cnt_v = jnp.zeros((L,), jnp.int32)
    acc_v = jnp.zeros((L,), jnp.int32)
    conf_v = jnp.zeros((L,), jnp.float32)
    for k in range(L):
        ca = plsc.load_gather(tabca, [lane * L + k])
        cnt_v = cnt_v + (ca & 0xFFFF)
        acc_v = acc_v + lax.shift_right_logical(ca, 16)
        conf_v = conf_v + plsc.load_gather(tabf, [lane * L + k])
    red[0] = cnt_v.astype(jnp.float32)
    red[1] = conf_v
    red[2] = acc_v.astype(jnp.float32)
    pltpu.sync_copy(red, out_hbm.at[wid])


_sc_hist = functools.partial(
    pl.kernel,
    mesh=plsc.VectorSubcoreMesh(
        core_axis_name="c", subcore_axis_name="s",
        num_cores=NC, num_subcores=NS),
    out_type=jax.ShapeDtypeStruct((NW, 3, L), jnp.float32),
    compiler_params=pltpu.CompilerParams(needs_layout_passes=False),
    scratch_types=(
        [pltpu.VMEM((CH,), jnp.float32),
         pltpu.VMEM((CH,), jnp.int32),
         pltpu.VMEM((CH,), jnp.int32)] * NBUF
        + [pltpu.VMEM((L * L,), jnp.int32),
           pltpu.VMEM((L * L,), jnp.float32)]
        + [pltpu.VMEM((3, L), jnp.float32)]
        + [pltpu.SemaphoreType.DMA] * NBUF
    ),
)(_sc_body)


def _fin_body(p_ref, o_ref):
    tot = jnp.sum(p_ref[...], axis=0)
    cnt = tot[0:1]
    sconf = tot[1:2]
    sacc = tot[2:3]
    denom = jnp.maximum(cnt, 1.0)
    contrib = jnp.abs(sconf / denom - sacc / denom) * (cnt / N_TOTAL)
    contrib = jnp.where(cnt > 0, contrib, 0.0)
    o_ref[0] = jnp.sum(contrib)


_finalize = pl.pallas_call(
    _fin_body,
    out_shape=jax.ShapeDtypeStruct((1,), jnp.float32),
    out_specs=pl.BlockSpec(memory_space=pltpu.SMEM),
)


def kernel(confidences, predictions, labels, title):
    return _finalize(_sc_hist(confidences, predictions, labels))
